# baseline (speedup 1.0000x reference)
import functools

import jax
import jax.numpy as jnp
from jax import lax
from jax.experimental import pallas as pl
from jax.experimental.pallas import tpu as pltpu
from jax.experimental.pallas import tpu_sc as plsc

OUT_D, OUT_H, OUT_W = 8, 8, 8
PTS = OUT_D * OUT_H * OUT_W
SCALE_INV = 4.0
NC, NS, LANES = 2, 16, 16
NW = NC * NS


def _f(v):
    return jnp.full((LANES,), v, dtype=jnp.float32)


def _i(v):
    return jnp.full((LANES,), v, dtype=jnp.int32)


def _bcast(ref2d, row, col):
    return plsc.load_gather(ref2d, [_i(row), _i(col)])


def _sc_detile(feat_map, bb, *, N, C, D, H, W, K):
    HW = H * W
    NSLICE = N * C * D
    per_tec = NSLICE // NW
    mesh = plsc.VectorSubcoreMesh(
        core_axis_name="c", subcore_axis_name="s", num_cores=NC, num_subcores=NS
    )

    @functools.partial(
        pl.kernel,
        out_type=jax.ShapeDtypeStruct((N * C * D * H * W,), jnp.float32),
        mesh=mesh,
        compiler_params=pltpu.CompilerParams(
            needs_layout_passes=False, use_tc_tiling_on_sc=True),
        scratch_types=[
            pltpu.VMEM((8, K), jnp.float32),
            pltpu.VMEM((H, W), jnp.float32),
            pltpu.VMEM((HW,), jnp.float32),
        ],
    )
    def detile_kernel(feat_ref, bbt_ref, out_ref, bbt_v, slab, flat):
        wid = lax.axis_index("s") * NC + lax.axis_index("c")
        pltpu.sync_copy(bbt_ref, bbt_v)

        dims = (float(W), float(H), float(D))

        @pl.loop(0, K // LANES, init_carry=jnp.float32(0.0))
        def any_live(g, carry):
            live = None
            for ax in range(3):
                lo = bbt_v[1 + ax, pl.ds(g * LANES, LANES)] * SCALE_INV
                hi = bbt_v[4 + ax, pl.ds(g * LANES, LANES)] * SCALE_INV
                dim = dims[ax]
                s = (hi - lo) / (dim - 1.0)
                t = (hi + lo + 1.0) / dim - 1.0
                ia = jnp.clip(((s * (-0.875) + t + 1.0) * dim - 1.0) * 0.5,
                              -1e4, 1e4)
                ib = jnp.clip(((s * 0.875 + t + 1.0) * dim - 1.0) * 0.5,
                              -1e4, 1e4)
                hi_s = jnp.maximum(ia, ib)
                lo_s = jnp.minimum(ia, ib)
                alive = (hi_s >= -1.5) & (lo_s < dim + 0.5)
                live = alive if live is None else (live & alive)
            return jnp.maximum(carry, jnp.max(jnp.where(live, 1.0, 0.0)))

        @pl.when(any_live > 0.5)
        def _copy():
            @pl.loop(0, per_tec)
            def _slice(i):
                sl = wid * per_tec + i
                d = sl % D
                nc = sl // D
                c = nc % C
                n = nc // C
                pltpu.sync_copy(feat_ref.at[n, c, d], slab)
                for r in range(H):
                    for c16 in range(W // LANES):
                        flat[pl.ds(r * W + c16 * LANES, LANES)] = (
                            slab[r, pl.ds(c16 * LANES, LANES)])
                pltpu.sync_copy(flat, out_ref.at[pl.ds(sl * HW, HW)])

    return detile_kernel(feat_map, bb)


def _pad8(bboxes):
    return jnp.pad(bboxes.astype(jnp.float32), ((0, 0), (0, 1))).T


def _sc_roi_align(tbl, bb, *, N, C, D, H, W, K):
    DH = D * H
    CDH = C * DH
    rois_per = K // NW
    mesh = plsc.VectorSubcoreMesh(
        core_axis_name="c", subcore_axis_name="s", num_cores=NC, num_subcores=NS
    )

    def bf16rt(x):
        u = plsc.bitcast(x, jnp.int32)
        r = (u + 0x7FFF + ((u >> 16) & 1)) & jnp.int32(-65536)
        return plsc.bitcast(r, jnp.float32)

    def ipos(o_f32, s, t, dim):
        g = (o_f32 * 2.0 + 1.0) / 8.0 - 1.0
        grid = s * g + t
        i = ((grid + 1.0) * float(dim) - 1.0) * 0.5
        return jnp.clip(i, -1e4, 1e4)

    def corners(o_i32, s, t, dim):
        i = ipos(o_i32.astype(jnp.float32), s, t, dim)
        ti = i.astype(jnp.int32)
        tf = ti.astype(jnp.float32)
        i0 = jnp.where(tf > i, ti - 1, ti)
        frac = i - i0.astype(jnp.float32)
        w1 = frac
        w0 = 1.0 - frac
        i1 = i0 + 1
        w0m = jnp.where((i0 >= 0) & (i0 < dim), w0, 0.0)
        w1m = jnp.where((i1 >= 0) & (i1 < dim), w1, 0.0)
        return (
            jnp.clip(i0, 0, dim - 1),
            jnp.clip(i1, 0, dim - 1),
            w0m,
            w1m,
        )

    @functools.partial(
        pl.kernel,
        out_type=jax.ShapeDtypeStruct((K * C * PTS,), jnp.float32),
        mesh=mesh,
        compiler_params=pltpu.CompilerParams(
            needs_layout_passes=False, use_tc_tiling_on_sc=False),
        scratch_types=[
            pltpu.VMEM((rois_per, 32), jnp.float32),
            pltpu.VMEM((C * PTS,), jnp.float32),
            pltpu.VMEM((C * PTS,), jnp.float32),
            pltpu.VMEM((2, 128, W), jnp.float32),
            pltpu.VMEM((2, 128), jnp.int32),
            pltpu.VMEM((2, 128), jnp.int32),
            pltpu.VMEM((32, 8, LANES), jnp.int32),
            pltpu.VMEM((32, 8, LANES), jnp.int32),
            pltpu.VMEM((32, 8, LANES), jnp.float32),
            pltpu.SemaphoreType.DMA,
            pltpu.SemaphoreType.DMA,
        ],
    )
    def sc_kernel(tbl_ref, bb_ref, out_ref, bb_v, zero_buf, out_buf, rows_buf,
                  bidx_buf, cidx_buf, ridx_b, xidx_b, wgt_b, sem_z, sem_g):
        wid = lax.axis_index("s") * NC + lax.axis_index("c")
        base_roi = wid * rois_per

        pltpu.sync_copy(bb_ref.at[pl.ds(base_roi, rois_per)], bb_v)

        zv = jnp.zeros((LANES,), jnp.float32)

        @pl.loop(0, (C * PTS) // (8 * LANES))
        def _zero(iz):
            for u in range(8):
                zero_buf[pl.ds(iz * 128 + u * 16, 16)] = zv

        zcopies = [
            pltpu.async_copy(
                zero_buf, out_ref.at[pl.ds((base_roi + j) * (C * PTS), C * PTS)],
                sem_z)
            for j in range(rois_per)
        ]
        for dsc in zcopies:
            dsc.wait()

        @pl.loop(0, rois_per)
        def _roi(j):
            k = base_roi + j
            x1 = _bcast(bb_v, j, 1) * SCALE_INV
            y1 = _bcast(bb_v, j, 2) * SCALE_INV
            z1 = _bcast(bb_v, j, 3) * SCALE_INV
            x2 = _bcast(bb_v, j, 4) * SCALE_INV
            y2 = _bcast(bb_v, j, 5) * SCALE_INV
            z2 = _bcast(bb_v, j, 6) * SCALE_INV
            nvec = _bcast(bb_v, j, 0).astype(jnp.int32)

            sx = bf16rt((x2 - x1) / float(W - 1))
            tx = bf16rt((x2 + x1 + 1.0) / float(W) - 1.0)
            sy = bf16rt((y2 - y1) / float(H - 1))
            ty = bf16rt((y2 + y1 + 1.0) / float(H) - 1.0)
            sz = bf16rt((z2 - z1) / float(D - 1))
            tz = bf16rt((z2 + z1 + 1.0) / float(D) - 1.0)

            def axis_live(s, t, dim):
                ia = ipos(_f(0.0), s, t, dim)
                ib = ipos(_f(7.0), s, t, dim)
                hi = jnp.max(jnp.maximum(ia, ib))
                lo = jnp.min(jnp.minimum(ia, ib))
                return (hi >= -1.0) & (lo < float(dim))

            live = (axis_live(sx, tx, W) & axis_live(sy, ty, H)
                    & axis_live(sz, tz, D))

            @pl.when(live)
            def _compute():
                ncdh = nvec * CDH

                for czy in range(4):
                    @pl.loop(0, 4)
                    def _bq(q, czy=czy):
                        zy = q * 16 + lax.iota(jnp.int32, 16)
                        zo = zy >> 3
                        yo = zy & 7
                        z0c, z1c, _, _ = corners(zo, sz, tz, D)
                        y0c, y1c, _, _ = corners(yo, sy, ty, H)
                        zc = z1c if (czy >> 1) else z0c
                        yc = y1c if (czy & 1) else y0c
                        rowbase = ncdh + zc * H + yc
                        half = czy >> 1
                        colb = (czy & 1) * 64 + q * 16
                        bidx_buf[half, pl.ds(colb, 16)] = rowbase

                @pl.loop(0, 32)
                def _gp(g):
                    p = g * 16 + lax.iota(jnp.int32, 16)
                    zo = p >> 6
                    yo = (p >> 3) & 7
                    xo = p & 7
                    zy = p >> 3
                    _, _, wz0, wz1 = corners(zo, sz, tz, D)
                    _, _, wy0, wy1 = corners(yo, sy, ty, H)
                    x0c, x1c, wx0, wx1 = corners(xo, sx, tx, W)
                    for a in range(8):
                        czy = a >> 1
                        dx = a & 1
                        wz = wz1 if (czy >> 1) else wz0
                        wy = wy1 if (czy & 1) else wy0
                        wx = wx1 if dx else wx0
                        wgt_b[g, a] = (wz * wy) * wx
                        ridx_b[g, a] = (czy & 1) * 64 + zy
                        xidx_b[g, a] = x1c if dx else x0c

                @pl.loop(0, C)
                def _ch(cix):
                    offv = jnp.full((LANES,), cix * DH, dtype=jnp.int32)
                    for hh in range(2):
                        @pl.loop(0, 8)
                        def _ci(q, hh=hh):
                            bv = bidx_buf[hh, pl.ds(q * 16, 16)]
                            cidx_buf[hh, pl.ds(q * 16, 16)] = bv + offv
                    g0 = pltpu.async_copy(
                        tbl_ref.at[cidx_buf.at[0]], rows_buf.at[0], sem_g)
                    g1 = pltpu.async_copy(
                        tbl_ref.at[cidx_buf.at[1]], rows_buf.at[1], sem_g)
                    g0.wait()
                    g1.wait()

                    @pl.loop(0, 32)
                    def _bl(g):
                        acc = jnp.zeros((LANES,), jnp.float32)
                        for a in range(8):
                            hv = _i(a >> 2)
                            rv = ridx_b[g, a]
                            xv = xidx_b[g, a]
                            w = wgt_b[g, a]
                            val = plsc.load_gather(rows_buf, [hv, rv, xv])
                            acc = acc + val * w
                        out_buf[pl.ds(cix * PTS + g * 16, 16)] = acc

                pltpu.sync_copy(out_buf, out_ref.at[pl.ds(k * (C * PTS), C * PTS)])

    return sc_kernel(tbl, bb)


def kernel(feat_map, bboxes):
    N, C, D, H, W = feat_map.shape
    K = bboxes.shape[0]
    bb = jnp.pad(bboxes.astype(jnp.float32), ((0, 0), (0, 32 - bboxes.shape[1])))
    flat_tbl = _sc_detile(feat_map, _pad8(bboxes), N=N, C=C, D=D, H=H, W=W, K=K)
    tbl = flat_tbl.reshape(N * C * D * H, W)
    out = _sc_roi_align(tbl, bb, N=N, C=C, D=D, H=H, W=W, K=K)
    return out.reshape(K, C, OUT_D, OUT_H, OUT_W)


# speedup vs baseline: 6.1243x; 6.1243x over previous
import functools

import jax
import jax.numpy as jnp
from jax import lax
from jax.experimental import pallas as pl
from jax.experimental.pallas import tpu as pltpu
from jax.experimental.pallas import tpu_sc as plsc

OUT_D, OUT_H, OUT_W = 8, 8, 8
PTS = OUT_D * OUT_H * OUT_W
SCALE_INV = 4.0
NC, NS, LANES = 2, 16, 16
NW = NC * NS


def _f(v):
    return jnp.full((LANES,), v, dtype=jnp.float32)


def _i(v):
    return jnp.full((LANES,), v, dtype=jnp.int32)


def _bcast(ref2d, row, col):
    return plsc.load_gather(ref2d, [_i(row), _i(col)])


def _sc_detile(feat_map, bb, *, N, C, D, H, W, K):
    HW = H * W
    NSLICE = N * C * D
    per_tec = NSLICE // NW
    mesh = plsc.VectorSubcoreMesh(
        core_axis_name="c", subcore_axis_name="s", num_cores=NC, num_subcores=NS
    )

    @functools.partial(
        pl.kernel,
        out_type=jax.ShapeDtypeStruct((N * C * D * H * W,), jnp.float32),
        mesh=mesh,
        compiler_params=pltpu.CompilerParams(
            needs_layout_passes=False, use_tc_tiling_on_sc=True),
        scratch_types=[
            pltpu.VMEM((8, K), jnp.float32),
            pltpu.VMEM((H, W), jnp.float32),
            pltpu.VMEM((HW,), jnp.float32),
        ],
    )
    def detile_kernel(feat_ref, bbt_ref, out_ref, bbt_v, slab, flat):
        wid = lax.axis_index("s") * NC + lax.axis_index("c")
        pltpu.sync_copy(bbt_ref, bbt_v)

        dims = (float(W), float(H), float(D))

        @pl.loop(0, K // LANES, init_carry=jnp.float32(0.0))
        def any_live(g, carry):
            live = None
            for ax in range(3):
                lo = bbt_v[1 + ax, pl.ds(g * LANES, LANES)] * SCALE_INV
                hi = bbt_v[4 + ax, pl.ds(g * LANES, LANES)] * SCALE_INV
                dim = dims[ax]
                s = (hi - lo) / (dim - 1.0)
                t = (hi + lo + 1.0) / dim - 1.0
                ia = jnp.clip(((s * (-0.875) + t + 1.0) * dim - 1.0) * 0.5,
                              -1e4, 1e4)
                ib = jnp.clip(((s * 0.875 + t + 1.0) * dim - 1.0) * 0.5,
                              -1e4, 1e4)
                hi_s = jnp.maximum(ia, ib)
                lo_s = jnp.minimum(ia, ib)
                alive = (hi_s >= -1.5) & (lo_s < dim + 0.5)
                live = alive if live is None else (live & alive)
            return jnp.maximum(carry, jnp.max(jnp.where(live, 1.0, 0.0)))

        @pl.when(any_live > 0.5)
        def _copy():
            @pl.loop(0, per_tec)
            def _slice(i):
                sl = wid * per_tec + i
                d = sl % D
                nc = sl // D
                c = nc % C
                n = nc // C
                pltpu.sync_copy(feat_ref.at[n, c, d], slab)
                for r in range(H):
                    for c16 in range(W // LANES):
                        flat[pl.ds(r * W + c16 * LANES, LANES)] = (
                            slab[r, pl.ds(c16 * LANES, LANES)])
                pltpu.sync_copy(flat, out_ref.at[pl.ds(sl * HW, HW)])

    return detile_kernel(feat_map, bb)


def _pad8(bboxes):
    return jnp.pad(bboxes.astype(jnp.float32), ((0, 0), (0, 1))).T


def _sc_roi_align(tbl, bb, *, N, C, D, H, W, K):
    DH = D * H
    CDH = C * DH
    rois_per = K // NW
    mesh = plsc.VectorSubcoreMesh(
        core_axis_name="c", subcore_axis_name="s", num_cores=NC, num_subcores=NS
    )

    def bf16rt(x):
        u = plsc.bitcast(x, jnp.int32)
        r = (u + 0x7FFF + ((u >> 16) & 1)) & jnp.int32(-65536)
        return plsc.bitcast(r, jnp.float32)

    def ipos(o_f32, s, t, dim):
        g = (o_f32 * 2.0 + 1.0) / 8.0 - 1.0
        grid = s * g + t
        i = ((grid + 1.0) * float(dim) - 1.0) * 0.5
        return jnp.clip(i, -1e4, 1e4)

    def corners(o_i32, s, t, dim):
        i = ipos(o_i32.astype(jnp.float32), s, t, dim)
        ti = i.astype(jnp.int32)
        tf = ti.astype(jnp.float32)
        i0 = jnp.where(tf > i, ti - 1, ti)
        frac = i - i0.astype(jnp.float32)
        w1 = frac
        w0 = 1.0 - frac
        i1 = i0 + 1
        w0m = jnp.where((i0 >= 0) & (i0 < dim), w0, 0.0)
        w1m = jnp.where((i1 >= 0) & (i1 < dim), w1, 0.0)
        return (
            jnp.clip(i0, 0, dim - 1),
            jnp.clip(i1, 0, dim - 1),
            w0m,
            w1m,
        )

    @functools.partial(
        pl.kernel,
        out_type=jax.ShapeDtypeStruct((K, C * PTS), jnp.float32),
        mesh=mesh,
        compiler_params=pltpu.CompilerParams(
            needs_layout_passes=False, use_tc_tiling_on_sc=False),
        scratch_types=[
            pltpu.VMEM((rois_per, 32), jnp.float32),
            pltpu.VMEM((C * PTS,), jnp.float32),
            pltpu.VMEM((C * PTS,), jnp.float32),
            pltpu.VMEM((2, 128, W), jnp.float32),
            pltpu.VMEM((2, 128), jnp.int32),
            pltpu.VMEM((2, 128), jnp.int32),
            pltpu.VMEM((32, 8, LANES), jnp.int32),
            pltpu.VMEM((32, 8, LANES), jnp.int32),
            pltpu.VMEM((32, 8, LANES), jnp.float32),
            pltpu.SemaphoreType.DMA,
            pltpu.SemaphoreType.DMA,
        ],
    )
    def sc_kernel(tbl_ref, bb_ref, out_ref, bb_v, zero_buf, out_buf, rows_buf,
                  bidx_buf, cidx_buf, ridx_b, xidx_b, wgt_b, sem_z, sem_g):
        wid = lax.axis_index("s") * NC + lax.axis_index("c")
        base_roi = wid * rois_per

        pltpu.sync_copy(bb_ref.at[pl.ds(base_roi, rois_per)], bb_v)

        zv = jnp.zeros((LANES,), jnp.float32)

        @pl.loop(0, (C * PTS) // (8 * LANES))
        def _zero(iz):
            for u in range(8):
                zero_buf[pl.ds(iz * 128 + u * 16, 16)] = zv

        zcopies = [
            pltpu.async_copy(zero_buf, out_ref.at[base_roi + j], sem_z)
            for j in range(rois_per)
        ]
        for dsc in zcopies:
            dsc.wait()

        @pl.loop(0, rois_per)
        def _roi(j):
            k = base_roi + j
            x1 = _bcast(bb_v, j, 1) * SCALE_INV
            y1 = _bcast(bb_v, j, 2) * SCALE_INV
            z1 = _bcast(bb_v, j, 3) * SCALE_INV
            x2 = _bcast(bb_v, j, 4) * SCALE_INV
            y2 = _bcast(bb_v, j, 5) * SCALE_INV
            z2 = _bcast(bb_v, j, 6) * SCALE_INV
            nvec = _bcast(bb_v, j, 0).astype(jnp.int32)

            sx = bf16rt((x2 - x1) / float(W - 1))
            tx = bf16rt((x2 + x1 + 1.0) / float(W) - 1.0)
            sy = bf16rt((y2 - y1) / float(H - 1))
            ty = bf16rt((y2 + y1 + 1.0) / float(H) - 1.0)
            sz = bf16rt((z2 - z1) / float(D - 1))
            tz = bf16rt((z2 + z1 + 1.0) / float(D) - 1.0)

            def axis_live(s, t, dim):
                ia = ipos(_f(0.0), s, t, dim)
                ib = ipos(_f(7.0), s, t, dim)
                hi = jnp.max(jnp.maximum(ia, ib))
                lo = jnp.min(jnp.minimum(ia, ib))
                return (hi >= -1.0) & (lo < float(dim))

            live = (axis_live(sx, tx, W) & axis_live(sy, ty, H)
                    & axis_live(sz, tz, D))

            @pl.when(live)
            def _compute():
                ncdh = nvec * CDH

                for czy in range(4):
                    @pl.loop(0, 4)
                    def _bq(q, czy=czy):
                        zy = q * 16 + lax.iota(jnp.int32, 16)
                        zo = zy >> 3
                        yo = zy & 7
                        z0c, z1c, _, _ = corners(zo, sz, tz, D)
                        y0c, y1c, _, _ = corners(yo, sy, ty, H)
                        zc = z1c if (czy >> 1) else z0c
                        yc = y1c if (czy & 1) else y0c
                        rowbase = ncdh + zc * H + yc
                        half = czy >> 1
                        colb = (czy & 1) * 64 + q * 16
                        bidx_buf[half, pl.ds(colb, 16)] = rowbase

                @pl.loop(0, 32)
                def _gp(g):
                    p = g * 16 + lax.iota(jnp.int32, 16)
                    zo = p >> 6
                    yo = (p >> 3) & 7
                    xo = p & 7
                    zy = p >> 3
                    _, _, wz0, wz1 = corners(zo, sz, tz, D)
                    _, _, wy0, wy1 = corners(yo, sy, ty, H)
                    x0c, x1c, wx0, wx1 = corners(xo, sx, tx, W)
                    for a in range(8):
                        czy = a >> 1
                        dx = a & 1
                        wz = wz1 if (czy >> 1) else wz0
                        wy = wy1 if (czy & 1) else wy0
                        wx = wx1 if dx else wx0
                        wgt_b[g, a] = (wz * wy) * wx
                        ridx_b[g, a] = (czy & 1) * 64 + zy
                        xidx_b[g, a] = x1c if dx else x0c

                @pl.loop(0, C)
                def _ch(cix):
                    offv = jnp.full((LANES,), cix * DH, dtype=jnp.int32)
                    for hh in range(2):
                        @pl.loop(0, 8)
                        def _ci(q, hh=hh):
                            bv = bidx_buf[hh, pl.ds(q * 16, 16)]
                            cidx_buf[hh, pl.ds(q * 16, 16)] = bv + offv
                    g0 = pltpu.async_copy(
                        tbl_ref.at[cidx_buf.at[0]], rows_buf.at[0], sem_g)
                    g1 = pltpu.async_copy(
                        tbl_ref.at[cidx_buf.at[1]], rows_buf.at[1], sem_g)
                    g0.wait()
                    g1.wait()

                    @pl.loop(0, 32)
                    def _bl(g):
                        acc = jnp.zeros((LANES,), jnp.float32)
                        for a in range(8):
                            hv = _i(a >> 2)
                            rv = ridx_b[g, a]
                            xv = xidx_b[g, a]
                            w = wgt_b[g, a]
                            val = plsc.load_gather(rows_buf, [hv, rv, xv])
                            acc = acc + val * w
                        out_buf[pl.ds(cix * PTS + g * 16, 16)] = acc

                pltpu.sync_copy(out_buf, out_ref.at[k])

    return sc_kernel(tbl, bb)


def kernel(feat_map, bboxes):
    N, C, D, H, W = feat_map.shape
    K = bboxes.shape[0]
    bb = jnp.pad(bboxes.astype(jnp.float32), ((0, 0), (0, 32 - bboxes.shape[1])))
    flat_tbl = _sc_detile(feat_map, _pad8(bboxes), N=N, C=C, D=D, H=H, W=W, K=K)
    tbl = flat_tbl.reshape(N * C * D * H, W)
    out = _sc_roi_align(tbl, bb, N=N, C=C, D=D, H=H, W=W, K=K)
    return out.reshape(K, C, OUT_D, OUT_H, OUT_W)


# speedup vs baseline: 14.9322x; 2.4382x over previous
import functools

import jax
import jax.numpy as jnp
from jax import lax
from jax.experimental import pallas as pl
from jax.experimental.pallas import tpu as pltpu
from jax.experimental.pallas import tpu_sc as plsc

OUT_D, OUT_H, OUT_W = 8, 8, 8
PTS = OUT_D * OUT_H * OUT_W
SCALE_INV = 4.0
NC, NS, LANES = 2, 16, 16
NW = NC * NS


def _f(v):
    return jnp.full((LANES,), v, dtype=jnp.float32)


def _i(v):
    return jnp.full((LANES,), v, dtype=jnp.int32)


def _bcast(ref2d, row, col):
    return plsc.load_gather(ref2d, [_i(row), _i(col)])


def _sc_detile(feat_map, bb, *, N, C, D, H, W, K):
    HW = H * W
    NSLICE = N * C * D
    per_tec = NSLICE // NW
    mesh = plsc.VectorSubcoreMesh(
        core_axis_name="c", subcore_axis_name="s", num_cores=NC, num_subcores=NS
    )

    @functools.partial(
        pl.kernel,
        out_type=jax.ShapeDtypeStruct((N * C * D * H * W,), jnp.float32),
        mesh=mesh,
        compiler_params=pltpu.CompilerParams(
            needs_layout_passes=False, use_tc_tiling_on_sc=True),
        scratch_types=[
            pltpu.VMEM((8, K), jnp.float32),
            pltpu.VMEM((H, W), jnp.float32),
            pltpu.VMEM((HW,), jnp.float32),
        ],
    )
    def detile_kernel(feat_ref, bbt_ref, out_ref, bbt_v, slab, flat):
        wid = lax.axis_index("s") * NC + lax.axis_index("c")
        pltpu.sync_copy(bbt_ref, bbt_v)

        dims = (float(W), float(H), float(D))

        @pl.loop(0, K // LANES, init_carry=jnp.float32(0.0))
        def any_live(g, carry):
            live = None
            for ax in range(3):
                lo = bbt_v[1 + ax, pl.ds(g * LANES, LANES)] * SCALE_INV
                hi = bbt_v[4 + ax, pl.ds(g * LANES, LANES)] * SCALE_INV
                dim = dims[ax]
                s = (hi - lo) / (dim - 1.0)
                t = (hi + lo + 1.0) / dim - 1.0
                ia = jnp.clip(((s * (-0.875) + t + 1.0) * dim - 1.0) * 0.5,
                              -1e4, 1e4)
                ib = jnp.clip(((s * 0.875 + t + 1.0) * dim - 1.0) * 0.5,
                              -1e4, 1e4)
                hi_s = jnp.maximum(ia, ib)
                lo_s = jnp.minimum(ia, ib)
                alive = (hi_s >= -1.5) & (lo_s < dim + 0.5)
                live = alive if live is None else (live & alive)
            return jnp.maximum(carry, jnp.max(jnp.where(live, 1.0, 0.0)))

        @pl.when(any_live > 0.5)
        def _copy():
            @pl.loop(0, per_tec)
            def _slice(i):
                sl = wid * per_tec + i
                d = sl % D
                nc = sl // D
                c = nc % C
                n = nc // C
                pltpu.sync_copy(feat_ref.at[n, c, d], slab)
                for r in range(H):
                    for c16 in range(W // LANES):
                        flat[pl.ds(r * W + c16 * LANES, LANES)] = (
                            slab[r, pl.ds(c16 * LANES, LANES)])
                pltpu.sync_copy(flat, out_ref.at[pl.ds(sl * HW, HW)])

    return detile_kernel(feat_map, bb)


def _pad8(bboxes):
    return jnp.pad(bboxes.astype(jnp.float32), ((0, 0), (0, 1))).T


def _sc_roi_align(tbl, bb, *, N, C, D, H, W, K):
    DH = D * H
    CDH = C * DH
    rois_per = K // NW
    mesh = plsc.VectorSubcoreMesh(
        core_axis_name="c", subcore_axis_name="s", num_cores=NC, num_subcores=NS
    )

    def bf16rt(x):
        u = plsc.bitcast(x, jnp.int32)
        r = (u + 0x7FFF + ((u >> 16) & 1)) & jnp.int32(-65536)
        return plsc.bitcast(r, jnp.float32)

    def ipos(o_f32, s, t, dim):
        g = (o_f32 * 2.0 + 1.0) / 8.0 - 1.0
        grid = s * g + t
        i = ((grid + 1.0) * float(dim) - 1.0) * 0.5
        return jnp.clip(i, -1e4, 1e4)

    def corners(o_i32, s, t, dim):
        i = ipos(o_i32.astype(jnp.float32), s, t, dim)
        ti = i.astype(jnp.int32)
        tf = ti.astype(jnp.float32)
        i0 = jnp.where(tf > i, ti - 1, ti)
        frac = i - i0.astype(jnp.float32)
        w1 = frac
        w0 = 1.0 - frac
        i1 = i0 + 1
        w0m = jnp.where((i0 >= 0) & (i0 < dim), w0, 0.0)
        w1m = jnp.where((i1 >= 0) & (i1 < dim), w1, 0.0)
        return (
            jnp.clip(i0, 0, dim - 1),
            jnp.clip(i1, 0, dim - 1),
            w0m,
            w1m,
        )

    @functools.partial(
        pl.kernel,
        out_type=jax.ShapeDtypeStruct((C * PTS // 8, 2, 8, 128), jnp.float32),
        mesh=mesh,
        compiler_params=pltpu.CompilerParams(
            needs_layout_passes=False, use_tc_tiling_on_sc=False),
        scratch_types=[
            pltpu.VMEM((K, 32), jnp.float32),
            pltpu.VMEM((16, 2, 8, 128), jnp.float32),
            pltpu.VMEM((K // LANES, LANES), jnp.float32),
            pltpu.VMEM((64, W), jnp.float32),
            pltpu.VMEM((64,), jnp.int32),
            pltpu.SemaphoreType.DMA,
            pltpu.SemaphoreType.DMA,
        ],
    )
    def sc_kernel(tbl_ref, bb_ref, out_ref, bb_v, chunk_buf, live_v,
                  rows_buf, cidx_buf, sem_z, sem_g):
        wid = lax.axis_index("s") * NC + lax.axis_index("c")

        pltpu.sync_copy(bb_ref, bb_v)

        zv = jnp.zeros((LANES,), jnp.float32)

        def zero_chunk_buf():
            @pl.loop(0, 16 * 2)
            def _zero(iz):
                for r in range(8):
                    for u in range(8):
                        chunk_buf[iz >> 1, iz & 1, r, pl.ds(u * 16, 16)] = zv

        zero_chunk_buf()

        lane0 = lax.iota(jnp.int32, LANES)

        @pl.loop(0, K // LANES, init_carry=jnp.float32(0.0))
        def any_live(gk, carry):
            rows = gk * LANES + lane0
            live = None
            for ax, dim in enumerate((W, H, D)):
                lo = plsc.load_gather(bb_v, [rows, _i(1 + ax)]) * SCALE_INV
                hi = plsc.load_gather(bb_v, [rows, _i(4 + ax)]) * SCALE_INV
                s = bf16rt((hi - lo) / float(dim - 1))
                t = bf16rt((hi + lo + 1.0) / float(dim) - 1.0)
                ia = ipos(_f(0.0), s, t, dim)
                ib = ipos(_f(7.0), s, t, dim)
                his = jnp.maximum(ia, ib)
                los = jnp.minimum(ia, ib)
                alive = (his >= -1.0) & (los < float(dim))
                live = alive if live is None else (live & alive)
            lf = jnp.where(live, 1.0, 0.0).astype(jnp.float32)
            live_v[gk] = lf
            return jnp.maximum(carry, jnp.max(lf))

        @pl.when(any_live < 0.5)
        def _fast():
            zcopies = [
                pltpu.async_copy(
                    chunk_buf, out_ref.at[pl.ds(wid * 128 + j * 16, 16)], sem_z)
                for j in range(8)
            ]
            for dsc in zcopies:
                dsc.wait()

        @pl.when(any_live >= 0.5)
        def _slow():
            lane = lax.iota(jnp.int32, LANES)

            @pl.loop(0, 8)
            def _chunk(j):
                c = wid * 2 + (j >> 2)
                qj = j & 3
                zy0 = qj * 16
                p0 = qj * 128

                @pl.loop(0, K)
                def _roi(k):
                    klive = plsc.load_gather(
                        live_v, [_i(0) + (k >> 4), _i(0) + (k & 15)])
                    @pl.when(jnp.max(klive) > 0.5)
                    def _do():
                        x1 = _bcast(bb_v, k, 1) * SCALE_INV
                        y1 = _bcast(bb_v, k, 2) * SCALE_INV
                        z1 = _bcast(bb_v, k, 3) * SCALE_INV
                        x2 = _bcast(bb_v, k, 4) * SCALE_INV
                        y2 = _bcast(bb_v, k, 5) * SCALE_INV
                        z2 = _bcast(bb_v, k, 6) * SCALE_INV
                        nvec = _bcast(bb_v, k, 0).astype(jnp.int32)
                        sx = bf16rt((x2 - x1) / float(W - 1))
                        tx = bf16rt((x2 + x1 + 1.0) / float(W) - 1.0)
                        sy = bf16rt((y2 - y1) / float(H - 1))
                        ty = bf16rt((y2 + y1 + 1.0) / float(H) - 1.0)
                        sz = bf16rt((z2 - z1) / float(D - 1))
                        tz = bf16rt((z2 + z1 + 1.0) / float(D) - 1.0)
                        base = nvec * CDH + c * DH

                        for czy in range(4):
                            zy = zy0 + lane
                            zo = zy >> 3
                            yo = zy & 7
                            z0c, z1c, _, _ = corners(zo, sz, tz, D)
                            y0c, y1c, _, _ = corners(yo, sy, ty, H)
                            zc = z1c if (czy >> 1) else z0c
                            yc = y1c if (czy & 1) else y0c
                            cidx_buf[pl.ds(czy * 16, 16)] = base + zc * H + yc

                        gat = pltpu.async_copy(
                            tbl_ref.at[cidx_buf], rows_buf, sem_g)
                        gat.wait()

                        ktv = _i(0) + (k >> 7)
                        kcv = _i(0) + (k & 127)
                        rv = lane & 7
                        for gl in range(8):
                            p = p0 + gl * 16 + lane
                            zo = p >> 6
                            yo = (p >> 3) & 7
                            xo = p & 7
                            zylo = (p >> 3) & 15
                            _, _, wz0, wz1 = corners(zo, sz, tz, D)
                            _, _, wy0, wy1 = corners(yo, sy, ty, H)
                            x0c, x1c, wx0, wx1 = corners(xo, sx, tx, W)
                            acc = jnp.zeros((LANES,), jnp.float32)
                            for a in range(8):
                                czy = a >> 1
                                dx = a & 1
                                wz = wz1 if (czy >> 1) else wz0
                                wy = wy1 if (czy & 1) else wy0
                                wx = wx1 if dx else wx0
                                w = (wz * wy) * wx
                                xv = x1c if dx else x0c
                                val = plsc.load_gather(
                                    rows_buf, [zylo + czy * 16, xv])
                                acc = acc + val * w
                            cptl = gl * 2 + (lane >> 3)
                            plsc.store_scatter(
                                chunk_buf, [cptl, ktv, rv, kcv], acc)

                pltpu.sync_copy(
                    chunk_buf, out_ref.at[pl.ds(wid * 128 + j * 16, 16)])
                zero_chunk_buf()

    return sc_kernel(tbl, bb)


def kernel(feat_map, bboxes):
    N, C, D, H, W = feat_map.shape
    K = bboxes.shape[0]
    bb = jnp.pad(bboxes.astype(jnp.float32), ((0, 0), (0, 32 - bboxes.shape[1])))
    flat_tbl = _sc_detile(feat_map, _pad8(bboxes), N=N, C=C, D=D, H=H, W=W, K=K)
    tbl = flat_tbl.reshape(N * C * D * H, W)
    out = _sc_roi_align(tbl, bb, N=N, C=C, D=D, H=H, W=W, K=K)
    return (out.transpose(1, 3, 0, 2)
            .reshape(K, C, OUT_D, OUT_H, OUT_W))
